# baseline (speedup 1.0000x reference)
import functools

import jax
import jax.numpy as jnp
from jax import lax
from jax.experimental import pallas as pl
from jax.experimental.pallas import tpu as pltpu
from jax.experimental.pallas import tpu_sc as plsc

_B = 8192
_EMBED = 768
_PDIM = 256
_SIZE = 64
_K = 8
_BLK = 512
_CH = 32


def _tc_body(query_ref, win_ref, pv_ref, wout_ref, idx_ref, loss_ref, p_ref):
    i = pl.program_id(0)

    q = lax.dot_general(query_ref[...], win_ref[...],
                        dimension_numbers=(((1,), (1,)), ((), ())),
                        preferred_element_type=jnp.float32)
    qn = q / jnp.maximum(
        jnp.sqrt(jnp.sum(q * q, axis=1, keepdims=True)), 1e-12)

    pv = pv_ref[...]
    keysn = pv / jnp.maximum(
        jnp.sqrt(jnp.sum(pv * pv, axis=1, keepdims=True)), 1e-12)

    sim = lax.dot_general(qn, keysn,
                          dimension_numbers=(((1,), (1,)), ((), ())),
                          preferred_element_type=jnp.float32)

    colid = lax.broadcasted_iota(jnp.int32, (_BLK, _SIZE), 1)
    work = sim
    mask = jnp.zeros((_BLK, _SIZE), jnp.bool_)
    idx_cols = []
    for _ in range(_K):
        m = jnp.max(work, axis=1, keepdims=True)
        cand = jnp.where(work == m, colid, _SIZE)
        sel = jnp.min(cand, axis=1, keepdims=True)
        onehot = colid == sel
        idx_cols.append(sel)
        mask = jnp.logical_or(mask, onehot)
        work = jnp.where(onehot, -jnp.inf, work)
    idx_ref[...] = jnp.concatenate(idx_cols, axis=1)

    recon = lax.dot_general(jnp.where(mask, sim, 0.0), keysn,
                            dimension_numbers=(((1,), (0,)), ((), ())),
                            preferred_element_type=jnp.float32)
    d = recon - qn
    diff_part = jnp.sum(d * d) * (1.0 / _B)

    @pl.when(i == 0)
    def _():
        g = lax.dot_general(keysn, keysn,
                            dimension_numbers=(((1,), (1,)), ((), ())),
                            preferred_element_type=jnp.float32)
        r = lax.broadcasted_iota(jnp.int32, (_SIZE, _SIZE), 0)
        c = lax.broadcasted_iota(jnp.int32, (_SIZE, _SIZE), 1)
        eye = (r == c).astype(jnp.float32)
        loss_ref[0, 0] = jnp.sum(jnp.abs(g - eye)) * (1.0 / _B)
        p_ref[...] = lax.dot_general(keysn, wout_ref[...],
                                     dimension_numbers=(((1,), (1,)), ((), ())),
                                     preferred_element_type=jnp.float32)

    loss_ref[0, 0] += diff_part


def _tc_stage(query2d, w_in, pv2d, w_out):
    grid = _B // _BLK
    return pl.pallas_call(
        _tc_body,
        grid=(grid,),
        in_specs=[
            pl.BlockSpec((_BLK, _EMBED), lambda i: (i, 0)),
            pl.BlockSpec((_PDIM, _EMBED), lambda i: (0, 0)),
            pl.BlockSpec((_SIZE, _PDIM), lambda i: (0, 0)),
            pl.BlockSpec((_EMBED, _PDIM), lambda i: (0, 0)),
        ],
        out_specs=[
            pl.BlockSpec((_BLK, _K), lambda i: (i, 0)),
            pl.BlockSpec((1, 1), lambda i: (0, 0),
                         memory_space=pltpu.MemorySpace.SMEM),
            pl.BlockSpec((_SIZE, _EMBED), lambda i: (0, 0)),
        ],
        out_shape=[
            jax.ShapeDtypeStruct((_B, _K), jnp.int32),
            jax.ShapeDtypeStruct((1, 1), jnp.float32),
            jax.ShapeDtypeStruct((_SIZE, _EMBED), jnp.float32),
        ],
    )(query2d, w_in, pv2d, w_out)


_NW = 32
_ROWS = _B * _K
_RPW = _ROWS // _NW
_NCH = _RPW // _CH
_NBUF = 4


_CG = _EMBED // 16


_CHW = _CH * _EMBED


def _sc_gather(p_flat, idx_flat):
    mesh = plsc.VectorSubcoreMesh(core_axis_name="c", subcore_axis_name="s")

    @functools.partial(
        pl.kernel,
        out_type=jax.ShapeDtypeStruct((_ROWS * _EMBED,), jnp.float32),
        mesh=mesh,
        compiler_params=pltpu.CompilerParams(needs_layout_passes=False),
        scratch_types=[
            pltpu.VMEM((_SIZE * _EMBED,), jnp.float32),
            pltpu.VMEM((_RPW,), jnp.int32),
            pltpu.VMEM((_CHW,), jnp.float32),
            pltpu.VMEM((_CHW,), jnp.float32),
            pltpu.SemaphoreType.DMA,
            pltpu.SemaphoreType.DMA,
        ],
    )
    def k(table_hbm, idx_hbm, out_hbm, table_v, idx_v, buf0, buf1,
          ssem0, ssem1):
        wid = lax.axis_index("s") * 2 + lax.axis_index("c")
        base = wid * _RPW
        bufs = (buf0, buf1)
        ssem = (ssem0, ssem1)
        pltpu.sync_copy(table_hbm, table_v)
        pltpu.sync_copy(idx_hbm.at[pl.ds(base, _RPW)], idx_v)
        lanes = lax.broadcasted_iota(jnp.int32, (16,), 0)

        def assemble(buf, ch):
            def row_body(p, carry):
                rid = plsc.load_gather(idx_v, [lanes * 0 + (ch * _CH + p)])
                rbase = rid * _EMBED + lanes
                obase = p * _EMBED
                for c in range(_CG):
                    vals = plsc.load_gather(table_v, [rbase + c * 16])
                    buf[pl.ds(obase + c * 16, 16)] = vals
                return carry
            lax.fori_loop(0, _CH, row_body, 0)

        def store(b, ch):
            return pltpu.make_async_copy(
                bufs[b],
                out_hbm.at[pl.ds((base + ch * _CH) * _EMBED, _CHW)],
                ssem[b])

        for b in range(2):
            assemble(bufs[b], b)
            store(b, b).start()

        def pair_body(i, carry):
            for b in range(2):
                ch = i * 2 + b
                store(b, ch - 2).wait()
                assemble(bufs[b], ch)
                store(b, ch).start()
            return carry
        lax.fori_loop(1, _NCH // 2, pair_body, 0)

        for ch in (_NCH - 2, _NCH - 1):
            store(ch % 2, ch).wait()

    return k(p_flat, idx_flat)


def kernel(query, W_in, prompt_values, W_out):
    query2d = query.reshape(_B, _EMBED)
    pv2d = prompt_values.reshape(_SIZE, _PDIM)
    idx, loss, p_table = _tc_stage(query2d, W_in, pv2d, W_out)
    rows = _sc_gather(p_table.reshape(_SIZE * _EMBED), idx.reshape(_ROWS))
    prompts_out = rows.reshape(_B, _K, _EMBED)
    return prompts_out, loss.reshape(1)


# speedup vs baseline: 1.5358x; 1.5358x over previous
import functools

import jax
import jax.numpy as jnp
from jax import lax
from jax.experimental import pallas as pl
from jax.experimental.pallas import tpu as pltpu
from jax.experimental.pallas import tpu_sc as plsc

_B = 8192
_EMBED = 768
_PDIM = 256
_SIZE = 64
_K = 8
_BLK = 512
_CH = 32


def _tc_body(query_ref, win_ref, pv_ref, wout_ref, idx_ref, loss_ref, p_ref):
    i = pl.program_id(0)

    q = lax.dot_general(query_ref[...], win_ref[...],
                        dimension_numbers=(((1,), (1,)), ((), ())),
                        preferred_element_type=jnp.float32)
    qn = q / jnp.maximum(
        jnp.sqrt(jnp.sum(q * q, axis=1, keepdims=True)), 1e-12)

    pv = pv_ref[...]
    keysn = pv / jnp.maximum(
        jnp.sqrt(jnp.sum(pv * pv, axis=1, keepdims=True)), 1e-12)

    sim = lax.dot_general(qn, keysn,
                          dimension_numbers=(((1,), (1,)), ((), ())),
                          preferred_element_type=jnp.float32)

    colid = lax.broadcasted_iota(jnp.int32, (_BLK, _SIZE), 1)
    work = sim
    mask = jnp.zeros((_BLK, _SIZE), jnp.bool_)
    idx_cols = []
    for _ in range(_K):
        m = jnp.max(work, axis=1, keepdims=True)
        cand = jnp.where(work == m, colid, _SIZE)
        sel = jnp.min(cand, axis=1, keepdims=True)
        onehot = colid == sel
        idx_cols.append(sel)
        mask = jnp.logical_or(mask, onehot)
        work = jnp.where(onehot, -jnp.inf, work)
    idx_ref[...] = jnp.concatenate(idx_cols, axis=1)

    recon = lax.dot_general(jnp.where(mask, sim, 0.0), keysn,
                            dimension_numbers=(((1,), (0,)), ((), ())),
                            preferred_element_type=jnp.float32)
    d = recon - qn
    diff_part = jnp.sum(d * d) * (1.0 / _B)

    @pl.when(i == 0)
    def _():
        g = lax.dot_general(keysn, keysn,
                            dimension_numbers=(((1,), (1,)), ((), ())),
                            preferred_element_type=jnp.float32)
        r = lax.broadcasted_iota(jnp.int32, (_SIZE, _SIZE), 0)
        c = lax.broadcasted_iota(jnp.int32, (_SIZE, _SIZE), 1)
        eye = (r == c).astype(jnp.float32)
        loss_ref[0, 0] = jnp.sum(jnp.abs(g - eye)) * (1.0 / _B)
        p_ref[...] = lax.dot_general(keysn, wout_ref[...],
                                     dimension_numbers=(((1,), (1,)), ((), ())),
                                     preferred_element_type=jnp.float32)

    loss_ref[0, 0] += diff_part


def _tc_stage(query2d, w_in, pv2d, w_out):
    grid = _B // _BLK
    return pl.pallas_call(
        _tc_body,
        grid=(grid,),
        in_specs=[
            pl.BlockSpec((_BLK, _EMBED), lambda i: (i, 0)),
            pl.BlockSpec((_PDIM, _EMBED), lambda i: (0, 0)),
            pl.BlockSpec((_SIZE, _PDIM), lambda i: (0, 0)),
            pl.BlockSpec((_EMBED, _PDIM), lambda i: (0, 0)),
        ],
        out_specs=[
            pl.BlockSpec((_BLK, _K), lambda i: (i, 0)),
            pl.BlockSpec((1, 1), lambda i: (0, 0),
                         memory_space=pltpu.MemorySpace.SMEM),
            pl.BlockSpec((_SIZE, _EMBED), lambda i: (0, 0)),
        ],
        out_shape=[
            jax.ShapeDtypeStruct((_B, _K), jnp.int32),
            jax.ShapeDtypeStruct((1, 1), jnp.float32),
            jax.ShapeDtypeStruct((_SIZE, _EMBED), jnp.float32),
        ],
    )(query2d, w_in, pv2d, w_out)


_NW = 32
_ROWS = _B * _K
_RPW = _ROWS // _NW
_NCH = _RPW // _CH
_NBUF = 4


_CG = _EMBED // 16


_CHW = _CH * _EMBED


def _sc_gather(p_flat, idx_flat):
    mesh = plsc.VectorSubcoreMesh(core_axis_name="c", subcore_axis_name="s")

    @functools.partial(
        pl.kernel,
        out_type=jax.ShapeDtypeStruct((_ROWS * _EMBED,), jnp.float32),
        mesh=mesh,
        compiler_params=pltpu.CompilerParams(needs_layout_passes=False),
        scratch_types=[
            pltpu.VMEM((_SIZE * _EMBED,), jnp.float32),
            pltpu.VMEM((_RPW,), jnp.int32),
            pltpu.VMEM((_CHW,), jnp.float32),
            pltpu.VMEM((_CHW,), jnp.float32),
            pltpu.SemaphoreType.DMA,
            pltpu.SemaphoreType.DMA,
        ],
    )
    def k(table_hbm, idx_hbm, out_hbm, table_v, idx_v, buf0, buf1,
          ssem0, ssem1):
        wid = lax.axis_index("s") * 2 + lax.axis_index("c")
        base = wid * _RPW
        bufs = (buf0, buf1)
        ssem = (ssem0, ssem1)
        pltpu.sync_copy(table_hbm, table_v)
        pltpu.sync_copy(idx_hbm.at[pl.ds(base, _RPW)], idx_v)
        lanes = lax.broadcasted_iota(jnp.int32, (16,), 0)

        def assemble(buf, ch):
            def row_body(p, carry):
                rid = plsc.load_gather(idx_v, [lanes * 0 + (ch * _CH + p)])
                rbase = rid * _EMBED + lanes
                obase = p * _EMBED
                for g in range(_CG // 8):
                    vs = [plsc.load_gather(table_v, [rbase + (g * 8 + u) * 16])
                          for u in range(8)]
                    for u in range(8):
                        buf[pl.ds(obase + (g * 8 + u) * 16, 16)] = vs[u]
                return carry
            lax.fori_loop(0, _CH, row_body, 0)

        def store(b, ch):
            return pltpu.make_async_copy(
                bufs[b],
                out_hbm.at[pl.ds((base + ch * _CH) * _EMBED, _CHW)],
                ssem[b])

        for b in range(2):
            assemble(bufs[b], b)
            store(b, b).start()

        def pair_body(i, carry):
            for b in range(2):
                ch = i * 2 + b
                store(b, ch - 2).wait()
                assemble(bufs[b], ch)
                store(b, ch).start()
            return carry
        lax.fori_loop(1, _NCH // 2, pair_body, 0)

        for ch in (_NCH - 2, _NCH - 1):
            store(ch % 2, ch).wait()

    return k(p_flat, idx_flat)


def kernel(query, W_in, prompt_values, W_out):
    query2d = query.reshape(_B, _EMBED)
    pv2d = prompt_values.reshape(_SIZE, _PDIM)
    idx, loss, p_table = _tc_stage(query2d, W_in, pv2d, W_out)
    rows = _sc_gather(p_table.reshape(_SIZE * _EMBED), idx.reshape(_ROWS))
    prompts_out = rows.reshape(_B, _K, _EMBED)
    return prompts_out, loss.reshape(1)


# speedup vs baseline: 2.8371x; 1.8473x over previous
import functools

import jax
import jax.numpy as jnp
from jax import lax
from jax.experimental import pallas as pl
from jax.experimental.pallas import tpu as pltpu
from jax.experimental.pallas import tpu_sc as plsc

_B = 8192
_EMBED = 768
_PDIM = 256
_SIZE = 64
_K = 8
_BLK = 512
_CH = 32


def _tc_body(query_ref, win_ref, pv_ref, wout_ref, idx_ref, loss_ref, p_ref):
    i = pl.program_id(0)

    q = lax.dot_general(query_ref[...], win_ref[...],
                        dimension_numbers=(((1,), (1,)), ((), ())),
                        preferred_element_type=jnp.float32)
    qn = q / jnp.maximum(
        jnp.sqrt(jnp.sum(q * q, axis=1, keepdims=True)), 1e-12)

    pv = pv_ref[...]
    keysn = pv / jnp.maximum(
        jnp.sqrt(jnp.sum(pv * pv, axis=1, keepdims=True)), 1e-12)

    sim = lax.dot_general(qn, keysn,
                          dimension_numbers=(((1,), (1,)), ((), ())),
                          preferred_element_type=jnp.float32)

    colid = lax.broadcasted_iota(jnp.int32, (_BLK, _SIZE), 1)
    work = sim
    mask = jnp.zeros((_BLK, _SIZE), jnp.bool_)
    idx_cols = []
    for _ in range(_K):
        m = jnp.max(work, axis=1, keepdims=True)
        cand = jnp.where(work == m, colid, _SIZE)
        sel = jnp.min(cand, axis=1, keepdims=True)
        onehot = colid == sel
        idx_cols.append(sel)
        mask = jnp.logical_or(mask, onehot)
        work = jnp.where(onehot, -jnp.inf, work)
    idx_ref[...] = jnp.concatenate(idx_cols, axis=1)

    recon = lax.dot_general(jnp.where(mask, sim, 0.0), keysn,
                            dimension_numbers=(((1,), (0,)), ((), ())),
                            preferred_element_type=jnp.float32)
    d = recon - qn
    diff_part = jnp.sum(d * d) * (1.0 / _B)

    @pl.when(i == 0)
    def _():
        g = lax.dot_general(keysn, keysn,
                            dimension_numbers=(((1,), (1,)), ((), ())),
                            preferred_element_type=jnp.float32)
        r = lax.broadcasted_iota(jnp.int32, (_SIZE, _SIZE), 0)
        c = lax.broadcasted_iota(jnp.int32, (_SIZE, _SIZE), 1)
        eye = (r == c).astype(jnp.float32)
        loss_ref[0, 0] = jnp.sum(jnp.abs(g - eye)) * (1.0 / _B)
        p_ref[...] = lax.dot_general(keysn, wout_ref[...],
                                     dimension_numbers=(((1,), (1,)), ((), ())),
                                     preferred_element_type=jnp.float32)

    loss_ref[0, 0] += diff_part


def _tc_stage(query2d, w_in, pv2d, w_out):
    grid = _B // _BLK
    return pl.pallas_call(
        _tc_body,
        grid=(grid,),
        in_specs=[
            pl.BlockSpec((_BLK, _EMBED), lambda i: (i, 0)),
            pl.BlockSpec((_PDIM, _EMBED), lambda i: (0, 0)),
            pl.BlockSpec((_SIZE, _PDIM), lambda i: (0, 0)),
            pl.BlockSpec((_EMBED, _PDIM), lambda i: (0, 0)),
        ],
        out_specs=[
            pl.BlockSpec((_BLK, _K), lambda i: (i, 0)),
            pl.BlockSpec((1, 1), lambda i: (0, 0),
                         memory_space=pltpu.MemorySpace.SMEM),
            pl.BlockSpec((_SIZE, _EMBED), lambda i: (0, 0)),
        ],
        out_shape=[
            jax.ShapeDtypeStruct((_B, _K), jnp.int32),
            jax.ShapeDtypeStruct((1, 1), jnp.float32),
            jax.ShapeDtypeStruct((_SIZE, _EMBED), jnp.float32),
        ],
    )(query2d, w_in, pv2d, w_out)


_NW = 32
_ROWS = _B * _K
_RPW = _ROWS // _NW
_NCH = _RPW // _CH
_NBUF = 4


_CG = _EMBED // 16


_CHW = _CH * _EMBED


def _sc_gather(p_flat, idx_flat):
    mesh = plsc.VectorSubcoreMesh(core_axis_name="c", subcore_axis_name="s")

    @functools.partial(
        pl.kernel,
        out_type=jax.ShapeDtypeStruct((_ROWS, _EMBED), jnp.float32),
        mesh=mesh,
        compiler_params=pltpu.CompilerParams(needs_layout_passes=False),
        scratch_types=[
            pltpu.VMEM((_SIZE * _EMBED,), jnp.float32),
            pltpu.VMEM((_RPW,), jnp.int32),
            pltpu.VMEM((_CH, _EMBED), jnp.float32),
            pltpu.VMEM((_CH, _EMBED), jnp.float32),
            pltpu.SemaphoreType.DMA,
            pltpu.SemaphoreType.DMA,
        ],
    )
    def k(table_hbm, idx_hbm, out_hbm, table_v, idx_v, buf0, buf1,
          ssem0, ssem1):
        wid = lax.axis_index("s") * 2 + lax.axis_index("c")
        base = wid * _RPW
        bufs = (buf0, buf1)
        ssem = (ssem0, ssem1)
        pltpu.sync_copy(table_hbm, table_v)
        pltpu.sync_copy(idx_hbm.at[pl.ds(base, _RPW)], idx_v)
        lanes = lax.broadcasted_iota(jnp.int32, (16,), 0)

        def assemble(buf, ch):
            def row_body(p, carry):
                rid = plsc.load_gather(idx_v, [lanes * 0 + (ch * _CH + p)])
                rbase = rid * _EMBED + lanes
                for g in range(_CG // 8):
                    vs = [plsc.load_gather(table_v, [rbase + (g * 8 + u) * 16])
                          for u in range(8)]
                    for u in range(8):
                        buf[p, pl.ds((g * 8 + u) * 16, 16)] = vs[u]
                return carry
            lax.fori_loop(0, _CH, row_body, 0)

        def store(b, ch):
            return pltpu.make_async_copy(
                bufs[b],
                out_hbm.at[pl.ds(base + ch * _CH, _CH)],
                ssem[b])

        for b in range(2):
            assemble(bufs[b], b)
            store(b, b).start()

        def pair_body(i, carry):
            for b in range(2):
                ch = i * 2 + b
                store(b, ch - 2).wait()
                assemble(bufs[b], ch)
                store(b, ch).start()
            return carry
        lax.fori_loop(1, _NCH // 2, pair_body, 0)

        for ch in (_NCH - 2, _NCH - 1):
            store(ch % 2, ch).wait()

    return k(p_flat, idx_flat)


def kernel(query, W_in, prompt_values, W_out):
    query2d = query.reshape(_B, _EMBED)
    pv2d = prompt_values.reshape(_SIZE, _PDIM)
    idx, loss, p_table = _tc_stage(query2d, W_in, pv2d, W_out)
    rows = _sc_gather(p_table.reshape(_SIZE * _EMBED), idx.reshape(_ROWS))
    prompts_out = rows.reshape(_B, _K, _EMBED)
    return prompts_out, loss.reshape(1)


# speedup vs baseline: 2.9531x; 1.0409x over previous
import functools

import jax
import jax.numpy as jnp
from jax import lax
from jax.experimental import pallas as pl
from jax.experimental.pallas import tpu as pltpu
from jax.experimental.pallas import tpu_sc as plsc

_B = 8192
_EMBED = 768
_PDIM = 256
_SIZE = 64
_K = 8
_BLK = 2048
_CH = 32


def _tc_body(query_ref, win_ref, pv_ref, wout_ref, idx_ref, loss_ref, p_ref):
    i = pl.program_id(0)

    q = lax.dot_general(query_ref[...], win_ref[...],
                        dimension_numbers=(((1,), (1,)), ((), ())),
                        preferred_element_type=jnp.float32)
    qn = q / jnp.maximum(
        jnp.sqrt(jnp.sum(q * q, axis=1, keepdims=True)), 1e-12)

    pv = pv_ref[...]
    keysn = pv / jnp.maximum(
        jnp.sqrt(jnp.sum(pv * pv, axis=1, keepdims=True)), 1e-12)

    sim = lax.dot_general(qn, keysn,
                          dimension_numbers=(((1,), (1,)), ((), ())),
                          preferred_element_type=jnp.float32)

    colid = lax.broadcasted_iota(jnp.int32, (_BLK, _SIZE), 1)
    work = sim
    mask = jnp.zeros((_BLK, _SIZE), jnp.bool_)
    idx_cols = []
    for _ in range(_K):
        m = jnp.max(work, axis=1, keepdims=True)
        cand = jnp.where(work == m, colid, _SIZE)
        sel = jnp.min(cand, axis=1, keepdims=True)
        onehot = colid == sel
        idx_cols.append(sel)
        mask = jnp.logical_or(mask, onehot)
        work = jnp.where(onehot, -jnp.inf, work)
    idx_ref[...] = jnp.concatenate(idx_cols, axis=1)

    recon = lax.dot_general(jnp.where(mask, sim, 0.0), keysn,
                            dimension_numbers=(((1,), (0,)), ((), ())),
                            preferred_element_type=jnp.float32)
    d = recon - qn
    diff_part = jnp.sum(d * d) * (1.0 / _B)

    @pl.when(i == 0)
    def _():
        g = lax.dot_general(keysn, keysn,
                            dimension_numbers=(((1,), (1,)), ((), ())),
                            preferred_element_type=jnp.float32)
        r = lax.broadcasted_iota(jnp.int32, (_SIZE, _SIZE), 0)
        c = lax.broadcasted_iota(jnp.int32, (_SIZE, _SIZE), 1)
        eye = (r == c).astype(jnp.float32)
        loss_ref[0, 0] = jnp.sum(jnp.abs(g - eye)) * (1.0 / _B)
        p_ref[...] = lax.dot_general(keysn, wout_ref[...],
                                     dimension_numbers=(((1,), (1,)), ((), ())),
                                     preferred_element_type=jnp.float32)

    loss_ref[0, 0] += diff_part


def _tc_stage(query2d, w_in, pv2d, w_out):
    grid = _B // _BLK
    return pl.pallas_call(
        _tc_body,
        grid=(grid,),
        in_specs=[
            pl.BlockSpec((_BLK, _EMBED), lambda i: (i, 0)),
            pl.BlockSpec((_PDIM, _EMBED), lambda i: (0, 0)),
            pl.BlockSpec((_SIZE, _PDIM), lambda i: (0, 0)),
            pl.BlockSpec((_EMBED, _PDIM), lambda i: (0, 0)),
        ],
        out_specs=[
            pl.BlockSpec((_BLK, _K), lambda i: (i, 0)),
            pl.BlockSpec((1, 1), lambda i: (0, 0),
                         memory_space=pltpu.MemorySpace.SMEM),
            pl.BlockSpec((_SIZE, _EMBED), lambda i: (0, 0)),
        ],
        out_shape=[
            jax.ShapeDtypeStruct((_B, _K), jnp.int32),
            jax.ShapeDtypeStruct((1, 1), jnp.float32),
            jax.ShapeDtypeStruct((_SIZE, _EMBED), jnp.float32),
        ],
    )(query2d, w_in, pv2d, w_out)


_NW = 32
_ROWS = _B * _K
_RPW = _ROWS // _NW
_NCH = _RPW // _CH
_NBUF = 4


_CG = _EMBED // 16


_CHW = _CH * _EMBED


def _sc_gather(p_flat, idx_flat):
    mesh = plsc.VectorSubcoreMesh(core_axis_name="c", subcore_axis_name="s")

    @functools.partial(
        pl.kernel,
        out_type=jax.ShapeDtypeStruct((_ROWS, _EMBED), jnp.float32),
        mesh=mesh,
        compiler_params=pltpu.CompilerParams(needs_layout_passes=False),
        scratch_types=[
            pltpu.VMEM((_SIZE * _EMBED,), jnp.float32),
            pltpu.VMEM((_RPW,), jnp.int32),
            pltpu.VMEM((_CH, _EMBED), jnp.float32),
            pltpu.VMEM((_CH, _EMBED), jnp.float32),
            pltpu.SemaphoreType.DMA,
            pltpu.SemaphoreType.DMA,
        ],
    )
    def k(table_hbm, idx_hbm, out_hbm, table_v, idx_v, buf0, buf1,
          ssem0, ssem1):
        wid = lax.axis_index("s") * 2 + lax.axis_index("c")
        base = wid * _RPW
        bufs = (buf0, buf1)
        ssem = (ssem0, ssem1)
        pltpu.sync_copy(table_hbm, table_v)
        pltpu.sync_copy(idx_hbm.at[pl.ds(base, _RPW)], idx_v)
        lanes = lax.broadcasted_iota(jnp.int32, (16,), 0)

        def assemble(buf, ch):
            def row_body(p, carry):
                rid = plsc.load_gather(idx_v, [lanes * 0 + (ch * _CH + p)])
                rbase = rid * _EMBED + lanes
                for g in range(_CG // 8):
                    vs = [plsc.load_gather(table_v, [rbase + (g * 8 + u) * 16])
                          for u in range(8)]
                    for u in range(8):
                        buf[p, pl.ds((g * 8 + u) * 16, 16)] = vs[u]
                return carry
            lax.fori_loop(0, _CH, row_body, 0)

        def store(b, ch):
            return pltpu.make_async_copy(
                bufs[b],
                out_hbm.at[pl.ds(base + ch * _CH, _CH)],
                ssem[b])

        for b in range(2):
            assemble(bufs[b], b)
            store(b, b).start()

        def pair_body(i, carry):
            for b in range(2):
                ch = i * 2 + b
                store(b, ch - 2).wait()
                assemble(bufs[b], ch)
                store(b, ch).start()
            return carry
        lax.fori_loop(1, _NCH // 2, pair_body, 0)

        for ch in (_NCH - 2, _NCH - 1):
            store(ch % 2, ch).wait()

    return k(p_flat, idx_flat)


def kernel(query, W_in, prompt_values, W_out):
    query2d = query.reshape(_B, _EMBED)
    pv2d = prompt_values.reshape(_SIZE, _PDIM)
    idx, loss, p_table = _tc_stage(query2d, W_in, pv2d, W_out)
    rows = _sc_gather(p_table.reshape(_SIZE * _EMBED), idx.reshape(_ROWS))
    prompts_out = rows.reshape(_B, _K, _EMBED)
    return prompts_out, loss.reshape(1)
